# baseline (speedup 1.0000x reference)
import functools

import jax
import jax.numpy as jnp
from jax import lax
from jax.experimental import pallas as pl
from jax.experimental.pallas import tpu as pltpu
from jax.experimental.pallas import tpu_sc as plsc

_SIG = [1.5 ** x for x in range(15)]
_NEG = 0.01

N = 10000
E = 160000
D = 128
PW = 256
NPAD = 10240
NC, NS = 2, 16
NW = NC * NS
EPAD = 163840
EPW = EPAD // NW
CH = 128
NCHUNK = EPW // CH
CHG = 128
NCHG = EPW // CHG


def _leaky(x):
    return jnp.where(x >= 0, x, _NEG * x)


def _mm(a, b):
    return lax.dot_general(a, b, (((1,), (0,)), ((), ())),
                           preferred_element_type=jnp.float32)


def _pack2(a, b):
    ia = lax.bitcast_convert_type(a, jnp.int32)
    ib = lax.bitcast_convert_type(b, jnp.int32)
    lo = jnp.right_shift(ia + 0x8000, 16) & 0xFFFF
    hi = (ib + 0x8000) & jnp.int32(-65536)
    return lo | hi


def _unpack2(g):
    a = lax.bitcast_convert_type(jnp.left_shift(g, 16), jnp.float32)
    b = lax.bitcast_convert_type(g & jnp.int32(-65536), jnp.float32)
    return a, b


def _proj_body(h_ref, c_ref, wa_ref, wb_ref, b1_ref, q_ref, k_ref, v_ref,
               pp_ref, qp_ref, oq_ref, ok_ref, ov_ref):
    h = h_ref[...]
    c = c_ref[...]
    z = jnp.zeros((h.shape[0], D - 3), jnp.float32)
    cz = jnp.concatenate([c, z], axis=1)
    pp_ref[...] = _pack2(_mm(h, wa_ref[...]) + b1_ref[...], cz)
    qp_ref[...] = _pack2(_mm(h, wb_ref[...]), -cz)
    oq_ref[...] = _leaky(_mm(h, q_ref[...])).astype(jnp.bfloat16)
    ok_ref[...] = _leaky(_mm(h, k_ref[...])).astype(jnp.bfloat16)
    ov_ref[...] = _mm(h, v_ref[...]).astype(jnp.bfloat16)


def _proj(h, coors, wa, wb, b1, attq, attk, attv):
    R = 400
    g = N // R
    wspec = pl.BlockSpec((D, D), lambda i: (0, 0))
    rspec = pl.BlockSpec((R, D), lambda i: (i, 0))
    pspec = pl.BlockSpec((R, D), lambda i: (i, 0))
    return pl.pallas_call(
        _proj_body,
        grid=(g,),
        in_specs=[rspec,
                  pl.BlockSpec((R, 3), lambda i: (i, 0)),
                  wspec, wspec,
                  pl.BlockSpec((1, D), lambda i: (0, 0)),
                  wspec, wspec, wspec],
        out_specs=[pspec, pspec, rspec, rspec, rspec],
        out_shape=[
            jax.ShapeDtypeStruct((N, D), jnp.int32),
            jax.ShapeDtypeStruct((N, D), jnp.int32),
            jax.ShapeDtypeStruct((N, D), jnp.bfloat16),
            jax.ShapeDtypeStruct((N, D), jnp.bfloat16),
            jax.ShapeDtypeStruct((N, D), jnp.bfloat16),
        ],
    )(h, coors, wa, wb, b1, attq, attk, attv)


def _fold_body(w2_ref, cw1_ref, b2_ref, cb1_ref, m_ref, bu_ref):
    m_ref[...] = _mm(w2_ref[...], cw1_ref[...])
    bu_ref[...] = _mm(b2_ref[...], cw1_ref[...]) + cb1_ref[...]


def _fold(w2, cw1, b2, cb1):
    return pl.pallas_call(
        _fold_body,
        out_shape=[
            jax.ShapeDtypeStruct((D, D), jnp.float32),
            jax.ShapeDtypeStruct((1, D), jnp.float32),
        ],
    )(w2, cw1, b2, cb1)


def _sc_gather_body(p_hbm, q_hbm, src_hbm, dst_hbm, gs_hbm, gd_hbm,
                    si_v, di_v, bs0_v, bd0_v, bs1_v, bd1_v,
                    sp0, sq0, sp1, sq1):
    wid = lax.axis_index("s") * NC + lax.axis_index("c")
    base = wid * EPW
    pltpu.sync_copy(src_hbm.at[pl.ds(base, EPW)], si_v)
    pltpu.sync_copy(dst_hbm.at[pl.ds(base, EPW)], di_v)

    def fire(c, bs, bd, sp, sq):
        off = c * CHG
        pltpu.async_copy(p_hbm.at[si_v.at[pl.ds(off, CHG)]], bs, sp)
        pltpu.async_copy(q_hbm.at[di_v.at[pl.ds(off, CHG)]], bd, sq)

    def drain_write(c, bs, bd, sp, sq):
        pltpu.make_async_copy(p_hbm.at[pl.ds(0, CHG)], bs, sp).wait()
        pltpu.make_async_copy(q_hbm.at[pl.ds(0, CHG)], bd, sq).wait()
        off = c * CHG
        pltpu.sync_copy(bs, gs_hbm.at[pl.ds(base + off, CHG)])
        pltpu.sync_copy(bd, gd_hbm.at[pl.ds(base + off, CHG)])

    fire(0, bs0_v, bd0_v, sp0, sq0)

    def stream_body(k, carry):
        c0 = 2 * k
        fire(c0 + 1, bs1_v, bd1_v, sp1, sq1)
        drain_write(c0, bs0_v, bd0_v, sp0, sq0)
        fire(jnp.minimum(c0 + 2, NCHG - 1), bs0_v, bd0_v, sp0, sq0)
        drain_write(c0 + 1, bs1_v, bd1_v, sp1, sq1)
        return carry

    lax.fori_loop(0, NCHG // 2, stream_body, 0)
    pltpu.make_async_copy(p_hbm.at[pl.ds(0, CHG)], bs0_v, sp0).wait()
    pltpu.make_async_copy(q_hbm.at[pl.ds(0, CHG)], bd0_v, sq0).wait()


def _sc_gather(ppack, qpack, src, dst):
    mesh = plsc.VectorSubcoreMesh(core_axis_name="c", subcore_axis_name="s")
    f = functools.partial(
        pl.kernel,
        mesh=mesh,
        compiler_params=pltpu.CompilerParams(needs_layout_passes=False),
        out_type=[
            jax.ShapeDtypeStruct((EPAD, D), jnp.int32),
            jax.ShapeDtypeStruct((EPAD, D), jnp.int32),
        ],
        scratch_types=[
            pltpu.VMEM((EPW,), jnp.int32),
            pltpu.VMEM((EPW,), jnp.int32),
            pltpu.VMEM((CHG, D), jnp.int32),
            pltpu.VMEM((CHG, D), jnp.int32),
            pltpu.VMEM((CHG, D), jnp.int32),
            pltpu.VMEM((CHG, D), jnp.int32),
            pltpu.SemaphoreType.DMA,
            pltpu.SemaphoreType.DMA,
            pltpu.SemaphoreType.DMA,
            pltpu.SemaphoreType.DMA,
        ],
    )(_sc_gather_body)
    return f(ppack, qpack, src, dst)


def _edge_body(gs_ref, gd_ref, w1c_ref, m_ref, bu_ref, cw2_ref,
               cb2_ref, t_ref, b_ref):
    i = pl.program_id(0)
    fa_s, fb_s = _unpack2(gs_ref[...])
    fa_d, fb_d = _unpack2(gd_ref[...])
    feat = fa_s + fa_d
    xrel = (fb_s + fb_d)[:, 0:3]
    r = feat.shape[0]
    mag = jnp.sum(xrel * xrel, axis=1, keepdims=True)
    rbf = jnp.concatenate([jnp.exp(-mag / s) for s in _SIG], axis=1)
    pre = feat + _mm(rbf, w1c_ref[...])
    t = _leaky(pre)
    u = _leaky(_mm(t, m_ref[...]) + bu_ref[...])
    coef = _mm(u, cw2_ref[...]) + cb2_ref[...]
    valid = (lax.broadcasted_iota(jnp.int32, (r, 1), 0) + i * r) < E
    t_ref[...] = jnp.where(valid, t, 0.0)
    xm = jnp.where(valid, xrel * coef, 0.0)
    cnt = jnp.where(valid, jnp.ones((r, 1), jnp.float32), 0.0)
    zpad = jnp.zeros((r, D - 4), jnp.float32)
    b_ref[...] = jnp.concatenate([xm, cnt, zpad], axis=1)


def _edge(gs, gd, w1c, m, bu, cw2, cb2):
    R = 8192
    g = EPAD // R
    return pl.pallas_call(
        _edge_body,
        grid=(g,),
        in_specs=[
            pl.BlockSpec((R, D), lambda i: (i, 0)),
            pl.BlockSpec((R, D), lambda i: (i, 0)),
            pl.BlockSpec((15, D), lambda i: (0, 0)),
            pl.BlockSpec((D, D), lambda i: (0, 0)),
            pl.BlockSpec((1, D), lambda i: (0, 0)),
            pl.BlockSpec((D, 1), lambda i: (0, 0)),
            pl.BlockSpec((1, 1), lambda i: (0, 0)),
        ],
        out_specs=[
            pl.BlockSpec((R, D), lambda i: (i, 0)),
            pl.BlockSpec((R, D), lambda i: (i, 0)),
        ],
        out_shape=[
            jax.ShapeDtypeStruct((EPAD, D), jnp.float32),
            jax.ShapeDtypeStruct((EPAD, D), jnp.float32),
        ],
    )(gs, gd, w1c, m, bu, cw2, cb2)


EPS = EPAD // NS
NCH2 = EPS // CH


def _sc_scatter_body(payt_hbm, payb_hbm, dst2_hbm, zero_hbm, out_hbm,
                     di_v, buf0_v, buf1_v, acc_sh, sl0, sl1):
    cid = lax.axis_index("c")
    sid = lax.axis_index("s")

    pltpu.sync_copy(zero_hbm.at[pl.ds(sid * 640, 640)],
                    acc_sh.at[pl.ds(sid * 640, 640)])
    pltpu.sync_copy(dst2_hbm.at[pl.ds(sid * NCH2, NCH2)], di_v)
    plsc.subcore_barrier()

    def run(pay_hbm):
        base = sid * EPS

        def fire(c, buf, sem):
            pltpu.async_copy(pay_hbm.at[pl.ds(base + c * CH, CH)], buf, sem)

        def drain_add(c, buf, sem):
            pltpu.make_async_copy(pay_hbm.at[pl.ds(0, CH)], buf, sem).wait()
            pltpu.sync_copy(buf, acc_sh.at[di_v.at[c]], add=True)

        fire(0, buf0_v, sl0)

        def body(k, carry):
            c0 = 2 * k
            fire(c0 + 1, buf1_v, sl1)
            drain_add(c0, buf0_v, sl0)
            fire(jnp.minimum(c0 + 2, NCH2 - 1), buf0_v, sl0)
            drain_add(c0 + 1, buf1_v, sl1)
            return carry

        lax.fori_loop(0, NCH2 // 2, body, 0)
        pltpu.make_async_copy(pay_hbm.at[pl.ds(0, CH)], buf0_v, sl0).wait()

    @pl.when(cid == 0)
    def _t():
        run(payt_hbm)

    @pl.when(cid == 1)
    def _b():
        run(payb_hbm)

    plsc.subcore_barrier()
    pltpu.sync_copy(acc_sh.at[pl.ds(sid * 640, 640)],
                    out_hbm.at[cid, pl.ds(sid * 640, 640)])


def _sc_scatter(pay_t, pay_b, dst2, zero):
    mesh = plsc.VectorSubcoreMesh(core_axis_name="c", subcore_axis_name="s")
    f = functools.partial(
        pl.kernel,
        mesh=mesh,
        compiler_params=pltpu.CompilerParams(needs_layout_passes=False),
        out_type=jax.ShapeDtypeStruct((NC, NPAD, D), jnp.float32),
        scratch_types=[
            pltpu.VMEM((NCH2, CH), jnp.int32),
            pltpu.VMEM((CH, D), jnp.float32),
            pltpu.VMEM((CH, D), jnp.float32),
            pltpu.VMEM_SHARED((NPAD, D), jnp.float32),
            pltpu.SemaphoreType.DMA,
            pltpu.SemaphoreType.DMA,
        ],
    )(_sc_scatter_body)
    return f(pay_t, pay_b, dst2, zero)


def _combine_body(acc_ref, w2_ref, b2_ref, x_ref, m_ref):
    a_t = acc_ref[0]
    a_b = acc_ref[1]
    cnt = a_b[:, 3:4]
    denom = jnp.maximum(cnt, 1.0)
    has = jnp.where(cnt > 0, 1.0, 0.0)
    x_ref[...] = a_b[:, 0:3] / denom
    m_ref[...] = _mm(a_t / denom, w2_ref[...]) + b2_ref[...] * has


def _combine(acc, w2, b2):
    R = 400
    g = N // R
    return pl.pallas_call(
        _combine_body,
        grid=(g,),
        in_specs=[
            pl.BlockSpec((NC, R, D), lambda i: (0, i, 0)),
            pl.BlockSpec((D, D), lambda i: (0, 0)),
            pl.BlockSpec((1, D), lambda i: (0, 0)),
        ],
        out_specs=[
            pl.BlockSpec((R, 3), lambda i: (i, 0)),
            pl.BlockSpec((R, D), lambda i: (i, 0)),
        ],
        out_shape=[
            jax.ShapeDtypeStruct((N, 3), jnp.float32),
            jax.ShapeDtypeStruct((N, D), jnp.float32),
        ],
    )(acc, w2, b2)


def _flash_body(q_ref, k_ref, v_ref, o_ref, m_scr, l_scr, acc_scr):
    j = pl.program_id(1)
    nj = pl.num_programs(1)

    @pl.when(j == 0)
    def _init():
        m_scr[...] = jnp.full_like(m_scr, -1e30)
        l_scr[...] = jnp.zeros_like(l_scr)
        acc_scr[...] = jnp.zeros_like(acc_scr)

    q = q_ref[...]
    k = k_ref[...]
    s = lax.dot_general(q, k, (((1,), (1,)), ((), ())),
                        preferred_element_type=jnp.float32)
    s = s.astype(jnp.float32)
    m_old = m_scr[:, :1]
    m_new = jnp.maximum(m_old, jnp.max(s, axis=1, keepdims=True))
    p = jnp.exp(s - m_new)
    corr = jnp.exp(m_old - m_new)
    l_new = l_scr[:, :1] * corr + jnp.sum(p, axis=1, keepdims=True)
    acc_scr[...] = acc_scr[...] * corr + _mm(p.astype(jnp.bfloat16),
                                             v_ref[...]).astype(jnp.float32)
    m_scr[...] = jnp.broadcast_to(m_new, m_scr.shape)
    l_scr[...] = jnp.broadcast_to(l_new, l_scr.shape)

    @pl.when(j == nj - 1)
    def _out():
        o_ref[...] = acc_scr[...] / l_scr[:, :1]


def _flash(q, k, v):
    BQ, BK = 1000, 2000
    gi, gj = N // BQ, N // BK
    return pl.pallas_call(
        _flash_body,
        grid=(gi, gj),
        in_specs=[
            pl.BlockSpec((BQ, D), lambda i, j: (i, 0)),
            pl.BlockSpec((BK, D), lambda i, j: (j, 0)),
            pl.BlockSpec((BK, D), lambda i, j: (j, 0)),
        ],
        out_specs=pl.BlockSpec((BQ, D), lambda i, j: (i, 0)),
        out_shape=jax.ShapeDtypeStruct((N, D), jnp.float32),
        scratch_shapes=[
            pltpu.VMEM((BQ, D), jnp.float32),
            pltpu.VMEM((BQ, D), jnp.float32),
            pltpu.VMEM((BQ, D), jnp.float32),
        ],
        compiler_params=pltpu.CompilerParams(
            dimension_semantics=("arbitrary", "arbitrary")),
    )(q, k, v)


def _asm_body(xl_ref, ml_ref, cl_ref, xr_ref, mr_ref, cr_ref, o_ref):
    o_ref[...] = jnp.concatenate(
        [xl_ref[...], ml_ref[...], cl_ref[...],
         xr_ref[...], mr_ref[...], cr_ref[...]], axis=1)


def _assemble(xl, ml, cl, xr, mr, cr):
    R = 400
    g = N // R
    x3 = pl.BlockSpec((R, 3), lambda i: (i, 0))
    fd = pl.BlockSpec((R, D), lambda i: (i, 0))
    return pl.pallas_call(
        _asm_body,
        grid=(g,),
        in_specs=[x3, fd, fd, x3, fd, fd],
        out_specs=pl.BlockSpec((R, 4 * D + 6), lambda i: (i, 0)),
        out_shape=jax.ShapeDtypeStruct((N, 4 * D + 6), jnp.float32),
    )(xl, ml, cl, xr, mr, cr)


def _per_graph(coors, h, ei, wa, wb, b1, w1c, m, bu, cw2, cb2, w2, b2,
               attq, attk, attv, zero, pad_i):
    ppack, qpack, oq, ok, ov = _proj(h, coors, wa, wb, b1, attq, attk, attv)
    src = jnp.concatenate([ei[0].astype(jnp.int32), pad_i])
    dst = jnp.concatenate([ei[1].astype(jnp.int32), pad_i])
    gs, gd = _sc_gather(ppack, qpack, src, dst)
    pay_t, pay_b = _edge(gs, gd, w1c, m, bu, cw2, cb2)
    dst2 = jnp.reshape(dst, (EPAD // CH, CH))
    acc = _sc_scatter(pay_t, pay_b, dst2, zero)
    x_upd, aggr = _combine(acc, w2, b2)
    return x_upd, aggr, oq, ok, ov


def kernel(coors_lig, h_feats_ligand, coors_rec, h_feats_receptor,
           edge_W1, edge_b1, edge_W2, edge_b2,
           att_Q, att_K, att_V,
           coors_W1, coors_b1, coors_W2, coors_b2,
           lig_edge_index, rec_edge_index):
    wa = edge_W1[:D]
    wb = edge_W1[D:2 * D]
    w1c = edge_W1[2 * D:]
    b1 = edge_b1.reshape(1, D)
    b2 = edge_b2.reshape(1, D)
    cb1 = coors_b1.reshape(1, D)
    cb2 = coors_b2.reshape(1, 1)
    m, bu = _fold(edge_W2, coors_W1, b2, cb1)
    zero = jnp.zeros((NPAD, D), jnp.float32)
    pad_i = jnp.zeros((EPAD - E,), jnp.int32)

    xl, ml, ql, kl, vl = _per_graph(
        coors_lig, h_feats_ligand, lig_edge_index,
        wa, wb, b1, w1c, m, bu, coors_W2, cb2, edge_W2, b2,
        att_Q, att_K, att_V, zero, pad_i)
    xr, mr, qr, kr, vr = _per_graph(
        coors_rec, h_feats_receptor, rec_edge_index,
        wa, wb, b1, w1c, m, bu, coors_W2, cb2, edge_W2, b2,
        att_Q, att_K, att_V, zero, pad_i)

    cl = _flash(ql, kr, vr)
    cr = _flash(qr, kl, vl)
    return _assemble(xl, ml, cl, xr, mr, cr)


# speedup vs baseline: 1.0864x; 1.0864x over previous
import functools

import jax
import jax.numpy as jnp
from jax import lax
from jax.experimental import pallas as pl
from jax.experimental.pallas import tpu as pltpu
from jax.experimental.pallas import tpu_sc as plsc

_SIG = [1.5 ** x for x in range(15)]
_NEG = 0.01

N = 10000
E = 160000
D = 128
PW = 256
NPAD = 10240
NC, NS = 2, 16
NW = NC * NS
EPAD = 163840
EPW = EPAD // NW
CH = 128
NCHUNK = EPW // CH
CHG = 128
NCHG = EPW // CHG


def _leaky(x):
    return jnp.where(x >= 0, x, _NEG * x)


def _mm(a, b):
    return lax.dot_general(a, b, (((1,), (0,)), ((), ())),
                           preferred_element_type=jnp.float32)


def _pack2(a, b):
    ia = lax.bitcast_convert_type(a, jnp.int32)
    ib = lax.bitcast_convert_type(b, jnp.int32)
    lo = jnp.right_shift(ia + 0x8000, 16) & 0xFFFF
    hi = (ib + 0x8000) & jnp.int32(-65536)
    return lo | hi


def _unpack2(g):
    a = lax.bitcast_convert_type(jnp.left_shift(g, 16), jnp.float32)
    b = lax.bitcast_convert_type(g & jnp.int32(-65536), jnp.float32)
    return a, b


def _proj_body(h_ref, c_ref, wa_ref, wb_ref, b1_ref, q_ref, k_ref, v_ref,
               pp_ref, qp_ref, oq_ref, ok_ref, ov_ref):
    h = h_ref[...]
    c = c_ref[...]
    z = jnp.zeros((h.shape[0], D - 3), jnp.float32)
    cz = jnp.concatenate([c, z], axis=1)
    pp_ref[...] = _pack2(_mm(h, wa_ref[...]) + b1_ref[...], cz)
    qp_ref[...] = _pack2(_mm(h, wb_ref[...]), -cz)
    oq_ref[...] = _leaky(_mm(h, q_ref[...])).astype(jnp.bfloat16)
    ok_ref[...] = _leaky(_mm(h, k_ref[...])).astype(jnp.bfloat16)
    ov_ref[...] = _mm(h, v_ref[...]).astype(jnp.bfloat16)


def _proj(h, coors, wa, wb, b1, attq, attk, attv):
    R = 400
    g = N // R
    wspec = pl.BlockSpec((D, D), lambda i: (0, 0))
    rspec = pl.BlockSpec((R, D), lambda i: (i, 0))
    pspec = pl.BlockSpec((R, D), lambda i: (i, 0))
    return pl.pallas_call(
        _proj_body,
        grid=(g,),
        in_specs=[rspec,
                  pl.BlockSpec((R, 3), lambda i: (i, 0)),
                  wspec, wspec,
                  pl.BlockSpec((1, D), lambda i: (0, 0)),
                  wspec, wspec, wspec],
        out_specs=[pspec, pspec, rspec, rspec, rspec],
        out_shape=[
            jax.ShapeDtypeStruct((N, D), jnp.int32),
            jax.ShapeDtypeStruct((N, D), jnp.int32),
            jax.ShapeDtypeStruct((N, D), jnp.bfloat16),
            jax.ShapeDtypeStruct((N, D), jnp.bfloat16),
            jax.ShapeDtypeStruct((N, D), jnp.bfloat16),
        ],
    )(h, coors, wa, wb, b1, attq, attk, attv)


def _fold_body(w2_ref, cw1_ref, b2_ref, cb1_ref, m_ref, bu_ref):
    m_ref[...] = _mm(w2_ref[...], cw1_ref[...])
    bu_ref[...] = _mm(b2_ref[...], cw1_ref[...]) + cb1_ref[...]


def _fold(w2, cw1, b2, cb1):
    return pl.pallas_call(
        _fold_body,
        out_shape=[
            jax.ShapeDtypeStruct((D, D), jnp.float32),
            jax.ShapeDtypeStruct((1, D), jnp.float32),
        ],
    )(w2, cw1, b2, cb1)


def _sc_gather_body(p_hbm, q_hbm, src_hbm, dst_hbm, gs_hbm, gd_hbm,
                    si_v, di_v, bs0_v, bd0_v, bs1_v, bd1_v,
                    sp0, sq0, sp1, sq1):
    wid = lax.axis_index("s") * NC + lax.axis_index("c")
    base = wid * EPW
    pltpu.sync_copy(src_hbm.at[pl.ds(base, EPW)], si_v)
    pltpu.sync_copy(dst_hbm.at[pl.ds(base, EPW)], di_v)

    def fire(c, bs, bd, sp, sq):
        off = c * CHG
        pltpu.async_copy(p_hbm.at[si_v.at[pl.ds(off, CHG)]], bs, sp)
        pltpu.async_copy(q_hbm.at[di_v.at[pl.ds(off, CHG)]], bd, sq)

    def drain_write(c, bs, bd, sp, sq):
        pltpu.make_async_copy(p_hbm.at[pl.ds(0, CHG)], bs, sp).wait()
        pltpu.make_async_copy(q_hbm.at[pl.ds(0, CHG)], bd, sq).wait()
        off = c * CHG
        pltpu.sync_copy(bs, gs_hbm.at[pl.ds(base + off, CHG)])
        pltpu.sync_copy(bd, gd_hbm.at[pl.ds(base + off, CHG)])

    fire(0, bs0_v, bd0_v, sp0, sq0)

    def stream_body(k, carry):
        c0 = 2 * k
        fire(c0 + 1, bs1_v, bd1_v, sp1, sq1)
        drain_write(c0, bs0_v, bd0_v, sp0, sq0)
        fire(jnp.minimum(c0 + 2, NCHG - 1), bs0_v, bd0_v, sp0, sq0)
        drain_write(c0 + 1, bs1_v, bd1_v, sp1, sq1)
        return carry

    lax.fori_loop(0, NCHG // 2, stream_body, 0)
    pltpu.make_async_copy(p_hbm.at[pl.ds(0, CHG)], bs0_v, sp0).wait()
    pltpu.make_async_copy(q_hbm.at[pl.ds(0, CHG)], bd0_v, sq0).wait()


def _sc_gather(ppack, qpack, src, dst):
    mesh = plsc.VectorSubcoreMesh(core_axis_name="c", subcore_axis_name="s")
    f = functools.partial(
        pl.kernel,
        mesh=mesh,
        compiler_params=pltpu.CompilerParams(needs_layout_passes=False),
        out_type=[
            jax.ShapeDtypeStruct((EPAD, D), jnp.int32),
            jax.ShapeDtypeStruct((EPAD, D), jnp.int32),
        ],
        scratch_types=[
            pltpu.VMEM((EPW,), jnp.int32),
            pltpu.VMEM((EPW,), jnp.int32),
            pltpu.VMEM((CHG, D), jnp.int32),
            pltpu.VMEM((CHG, D), jnp.int32),
            pltpu.VMEM((CHG, D), jnp.int32),
            pltpu.VMEM((CHG, D), jnp.int32),
            pltpu.SemaphoreType.DMA,
            pltpu.SemaphoreType.DMA,
            pltpu.SemaphoreType.DMA,
            pltpu.SemaphoreType.DMA,
        ],
    )(_sc_gather_body)
    return f(ppack, qpack, src, dst)


def _edge_body(gs_ref, gd_ref, w1c_ref, m_ref, bu_ref, cw2_ref,
               cb2_ref, t_ref, b_ref):
    i = pl.program_id(0)
    fa_s, fb_s = _unpack2(gs_ref[...])
    fa_d, fb_d = _unpack2(gd_ref[...])
    feat = fa_s + fa_d
    xrel = (fb_s + fb_d)[:, 0:3]
    r = feat.shape[0]
    mag = jnp.sum(xrel * xrel, axis=1, keepdims=True)
    rbf = jnp.concatenate([jnp.exp(-mag / s) for s in _SIG], axis=1)
    pre = feat + _mm(rbf, w1c_ref[...])
    t = _leaky(pre)
    u = _leaky(_mm(t, m_ref[...]) + bu_ref[...])
    coef = _mm(u, cw2_ref[...]) + cb2_ref[...]
    valid = (lax.broadcasted_iota(jnp.int32, (r, 1), 0) + i * r) < E
    t_ref[...] = jnp.where(valid, t, 0.0)
    xm = jnp.where(valid, xrel * coef, 0.0)
    cnt = jnp.where(valid, jnp.ones((r, 1), jnp.float32), 0.0)
    zpad = jnp.zeros((r, D - 4), jnp.float32)
    b_ref[...] = jnp.concatenate([xm, cnt, zpad], axis=1)


def _edge(gs, gd, w1c, m, bu, cw2, cb2):
    R = 8192
    g = EPAD // R
    return pl.pallas_call(
        _edge_body,
        grid=(g,),
        in_specs=[
            pl.BlockSpec((R, D), lambda i: (i, 0)),
            pl.BlockSpec((R, D), lambda i: (i, 0)),
            pl.BlockSpec((15, D), lambda i: (0, 0)),
            pl.BlockSpec((D, D), lambda i: (0, 0)),
            pl.BlockSpec((1, D), lambda i: (0, 0)),
            pl.BlockSpec((D, 1), lambda i: (0, 0)),
            pl.BlockSpec((1, 1), lambda i: (0, 0)),
        ],
        out_specs=[
            pl.BlockSpec((R, D), lambda i: (i, 0)),
            pl.BlockSpec((R, D), lambda i: (i, 0)),
        ],
        out_shape=[
            jax.ShapeDtypeStruct((EPAD, D), jnp.float32),
            jax.ShapeDtypeStruct((EPAD, D), jnp.float32),
        ],
    )(gs, gd, w1c, m, bu, cw2, cb2)


def _sc_scatter_body(pay_hbm, dst2_hbm, zero_hbm, out_hbm,
                     di_v, buf0_v, buf1_v, acc_sh, sl0, sl1):
    cid = lax.axis_index("c")
    sid = lax.axis_index("s")
    wid = sid * NC + cid
    base = wid * EPW

    pltpu.sync_copy(zero_hbm.at[pl.ds(sid * 640, 640)],
                    acc_sh.at[pl.ds(sid * 640, 640)])
    pltpu.sync_copy(dst2_hbm.at[pl.ds(wid * NCHUNK, NCHUNK)], di_v)
    plsc.subcore_barrier()

    def fire(c, buf, sem):
        pltpu.async_copy(pay_hbm.at[pl.ds(base + c * CH, CH)], buf, sem)

    def drain_add(c, buf, sem):
        pltpu.make_async_copy(pay_hbm.at[pl.ds(0, CH)], buf, sem).wait()
        pltpu.sync_copy(buf, acc_sh.at[di_v.at[c]], add=True)

    fire(0, buf0_v, sl0)

    def body(k, carry):
        c0 = 2 * k
        fire(c0 + 1, buf1_v, sl1)
        drain_add(c0, buf0_v, sl0)
        fire(jnp.minimum(c0 + 2, NCHUNK - 1), buf0_v, sl0)
        drain_add(c0 + 1, buf1_v, sl1)
        return carry

    lax.fori_loop(0, NCHUNK // 2, body, 0)
    pltpu.make_async_copy(pay_hbm.at[pl.ds(0, CH)], buf0_v, sl0).wait()
    plsc.subcore_barrier()
    pltpu.sync_copy(acc_sh.at[pl.ds(sid * 640, 640)],
                    out_hbm.at[cid, pl.ds(sid * 640, 640)])


def _sc_scatter(pay, dst2, zero):
    mesh = plsc.VectorSubcoreMesh(core_axis_name="c", subcore_axis_name="s")
    f = functools.partial(
        pl.kernel,
        mesh=mesh,
        compiler_params=pltpu.CompilerParams(needs_layout_passes=False),
        out_type=jax.ShapeDtypeStruct((NC, NPAD, D), jnp.float32),
        scratch_types=[
            pltpu.VMEM((NCHUNK, CH), jnp.int32),
            pltpu.VMEM((CH, D), jnp.float32),
            pltpu.VMEM((CH, D), jnp.float32),
            pltpu.VMEM_SHARED((NPAD, D), jnp.float32),
            pltpu.SemaphoreType.DMA,
            pltpu.SemaphoreType.DMA,
        ],
    )(_sc_scatter_body)
    return f(pay, dst2, zero)


def _combine_body(at_ref, ab_ref, w2_ref, b2_ref, x_ref, m_ref):
    a_t = at_ref[0] + at_ref[1]
    a_b = ab_ref[0] + ab_ref[1]
    cnt = a_b[:, 3:4]
    denom = jnp.maximum(cnt, 1.0)
    has = jnp.where(cnt > 0, 1.0, 0.0)
    x_ref[...] = a_b[:, 0:3] / denom
    m_ref[...] = _mm(a_t / denom, w2_ref[...]) + b2_ref[...] * has


def _combine(acc_t, acc_b, w2, b2):
    R = 400
    g = N // R
    return pl.pallas_call(
        _combine_body,
        grid=(g,),
        in_specs=[
            pl.BlockSpec((NC, R, D), lambda i: (0, i, 0)),
            pl.BlockSpec((NC, R, D), lambda i: (0, i, 0)),
            pl.BlockSpec((D, D), lambda i: (0, 0)),
            pl.BlockSpec((1, D), lambda i: (0, 0)),
        ],
        out_specs=[
            pl.BlockSpec((R, 3), lambda i: (i, 0)),
            pl.BlockSpec((R, D), lambda i: (i, 0)),
        ],
        out_shape=[
            jax.ShapeDtypeStruct((N, 3), jnp.float32),
            jax.ShapeDtypeStruct((N, D), jnp.float32),
        ],
    )(acc_t, acc_b, w2, b2)


def _flash_body(q_ref, k_ref, v_ref, o_ref, m_scr, l_scr, acc_scr):
    j = pl.program_id(1)
    nj = pl.num_programs(1)

    @pl.when(j == 0)
    def _init():
        m_scr[...] = jnp.full_like(m_scr, -1e30)
        l_scr[...] = jnp.zeros_like(l_scr)
        acc_scr[...] = jnp.zeros_like(acc_scr)

    q = q_ref[...]
    k = k_ref[...]
    s = lax.dot_general(q, k, (((1,), (1,)), ((), ())),
                        preferred_element_type=jnp.float32)
    s = s.astype(jnp.float32)
    m_old = m_scr[:, :1]
    m_new = jnp.maximum(m_old, jnp.max(s, axis=1, keepdims=True))
    p = jnp.exp(s - m_new)
    corr = jnp.exp(m_old - m_new)
    l_new = l_scr[:, :1] * corr + jnp.sum(p, axis=1, keepdims=True)
    acc_scr[...] = acc_scr[...] * corr + _mm(p.astype(jnp.bfloat16),
                                             v_ref[...]).astype(jnp.float32)
    m_scr[...] = jnp.broadcast_to(m_new, m_scr.shape)
    l_scr[...] = jnp.broadcast_to(l_new, l_scr.shape)

    @pl.when(j == nj - 1)
    def _out():
        o_ref[...] = acc_scr[...] / l_scr[:, :1]


def _flash(q, k, v):
    BQ, BK = 1000, 2000
    gi, gj = N // BQ, N // BK
    return pl.pallas_call(
        _flash_body,
        grid=(gi, gj),
        in_specs=[
            pl.BlockSpec((BQ, D), lambda i, j: (i, 0)),
            pl.BlockSpec((BK, D), lambda i, j: (j, 0)),
            pl.BlockSpec((BK, D), lambda i, j: (j, 0)),
        ],
        out_specs=pl.BlockSpec((BQ, D), lambda i, j: (i, 0)),
        out_shape=jax.ShapeDtypeStruct((N, D), jnp.float32),
        scratch_shapes=[
            pltpu.VMEM((BQ, D), jnp.float32),
            pltpu.VMEM((BQ, D), jnp.float32),
            pltpu.VMEM((BQ, D), jnp.float32),
        ],
        compiler_params=pltpu.CompilerParams(
            dimension_semantics=("arbitrary", "arbitrary")),
    )(q, k, v)


def _asm_body(xl_ref, ml_ref, cl_ref, xr_ref, mr_ref, cr_ref, o_ref):
    o_ref[...] = jnp.concatenate(
        [xl_ref[...], ml_ref[...], cl_ref[...],
         xr_ref[...], mr_ref[...], cr_ref[...]], axis=1)


def _assemble(xl, ml, cl, xr, mr, cr):
    R = 400
    g = N // R
    x3 = pl.BlockSpec((R, 3), lambda i: (i, 0))
    fd = pl.BlockSpec((R, D), lambda i: (i, 0))
    return pl.pallas_call(
        _asm_body,
        grid=(g,),
        in_specs=[x3, fd, fd, x3, fd, fd],
        out_specs=pl.BlockSpec((R, 4 * D + 6), lambda i: (i, 0)),
        out_shape=jax.ShapeDtypeStruct((N, 4 * D + 6), jnp.float32),
    )(xl, ml, cl, xr, mr, cr)


def _per_graph(coors, h, ei, wa, wb, b1, w1c, m, bu, cw2, cb2, w2, b2,
               attq, attk, attv, zero, pad_i):
    ppack, qpack, oq, ok, ov = _proj(h, coors, wa, wb, b1, attq, attk, attv)
    src = jnp.concatenate([ei[0].astype(jnp.int32), pad_i])
    dst = jnp.concatenate([ei[1].astype(jnp.int32), pad_i])
    gs, gd = _sc_gather(ppack, qpack, src, dst)
    pay_t, pay_b = _edge(gs, gd, w1c, m, bu, cw2, cb2)
    dst2 = jnp.reshape(dst, (EPAD // CH, CH))
    acc_t = _sc_scatter(pay_t, dst2, zero)
    acc_b = _sc_scatter(pay_b, dst2, zero)
    x_upd, aggr = _combine(acc_t, acc_b, w2, b2)
    return x_upd, aggr, oq, ok, ov


def kernel(coors_lig, h_feats_ligand, coors_rec, h_feats_receptor,
           edge_W1, edge_b1, edge_W2, edge_b2,
           att_Q, att_K, att_V,
           coors_W1, coors_b1, coors_W2, coors_b2,
           lig_edge_index, rec_edge_index):
    wa = edge_W1[:D]
    wb = edge_W1[D:2 * D]
    w1c = edge_W1[2 * D:]
    b1 = edge_b1.reshape(1, D)
    b2 = edge_b2.reshape(1, D)
    cb1 = coors_b1.reshape(1, D)
    cb2 = coors_b2.reshape(1, 1)
    m, bu = _fold(edge_W2, coors_W1, b2, cb1)
    zero = jnp.zeros((NPAD, D), jnp.float32)
    pad_i = jnp.zeros((EPAD - E,), jnp.int32)

    xl, ml, ql, kl, vl = _per_graph(
        coors_lig, h_feats_ligand, lig_edge_index,
        wa, wb, b1, w1c, m, bu, coors_W2, cb2, edge_W2, b2,
        att_Q, att_K, att_V, zero, pad_i)
    xr, mr, qr, kr, vr = _per_graph(
        coors_rec, h_feats_receptor, rec_edge_index,
        wa, wb, b1, w1c, m, bu, coors_W2, cb2, edge_W2, b2,
        att_Q, att_K, att_V, zero, pad_i)

    cl = _flash(ql, kr, vr)
    cr = _flash(qr, kl, vl)
    return _assemble(xl, ml, cl, xr, mr, cr)


# speedup vs baseline: 1.0953x; 1.0082x over previous
import functools

import jax
import jax.numpy as jnp
from jax import lax
from jax.experimental import pallas as pl
from jax.experimental.pallas import tpu as pltpu
from jax.experimental.pallas import tpu_sc as plsc

_SIG = [1.5 ** x for x in range(15)]
_NEG = 0.01

N = 10000
E = 160000
D = 128
PW = 256
NPAD = 10240
NC, NS = 2, 16
NW = NC * NS
EPAD = 163840
EPW = EPAD // NW
CH = 128
NCHUNK = EPW // CH
CHG = 128
NCHG = EPW // CHG


def _leaky(x):
    return jnp.where(x >= 0, x, _NEG * x)


def _mm(a, b):
    return lax.dot_general(a, b, (((1,), (0,)), ((), ())),
                           preferred_element_type=jnp.float32)


def _pack2(a, b):
    ia = lax.bitcast_convert_type(a, jnp.int32)
    ib = lax.bitcast_convert_type(b, jnp.int32)
    lo = jnp.right_shift(ia + 0x8000, 16) & 0xFFFF
    hi = (ib + 0x8000) & jnp.int32(-65536)
    return lo | hi


def _unpack2(g):
    a = lax.bitcast_convert_type(jnp.left_shift(g, 16), jnp.float32)
    b = lax.bitcast_convert_type(g & jnp.int32(-65536), jnp.float32)
    return a, b


def _proj_body(h_ref, c_ref, wa_ref, wb_ref, b1_ref, q_ref, k_ref, v_ref,
               pp_ref, qp_ref, oq_ref, ok_ref, ov_ref):
    h = h_ref[...]
    c = c_ref[...]
    z = jnp.zeros((h.shape[0], D - 3), jnp.float32)
    cz = jnp.concatenate([c, z], axis=1)
    pp_ref[...] = _pack2(_mm(h, wa_ref[...]) + b1_ref[...], cz)
    qp_ref[...] = _pack2(_mm(h, wb_ref[...]), -cz)
    oq_ref[...] = _leaky(_mm(h, q_ref[...])).astype(jnp.bfloat16)
    ok_ref[...] = _leaky(_mm(h, k_ref[...])).astype(jnp.bfloat16)
    ov_ref[...] = _mm(h, v_ref[...]).astype(jnp.bfloat16)


def _proj(h, coors, wa, wb, b1, attq, attk, attv):
    R = 400
    g = N // R
    wspec = pl.BlockSpec((D, D), lambda i: (0, 0))
    rspec = pl.BlockSpec((R, D), lambda i: (i, 0))
    pspec = pl.BlockSpec((R, D), lambda i: (i, 0))
    return pl.pallas_call(
        _proj_body,
        grid=(g,),
        in_specs=[rspec,
                  pl.BlockSpec((R, 3), lambda i: (i, 0)),
                  wspec, wspec,
                  pl.BlockSpec((1, D), lambda i: (0, 0)),
                  wspec, wspec, wspec],
        out_specs=[pspec, pspec, rspec, rspec, rspec],
        out_shape=[
            jax.ShapeDtypeStruct((N, D), jnp.int32),
            jax.ShapeDtypeStruct((N, D), jnp.int32),
            jax.ShapeDtypeStruct((N, D), jnp.bfloat16),
            jax.ShapeDtypeStruct((N, D), jnp.bfloat16),
            jax.ShapeDtypeStruct((N, D), jnp.bfloat16),
        ],
    )(h, coors, wa, wb, b1, attq, attk, attv)


def _fold_body(w2_ref, cw1_ref, b2_ref, cb1_ref, m_ref, bu_ref):
    m_ref[...] = _mm(w2_ref[...], cw1_ref[...])
    bu_ref[...] = _mm(b2_ref[...], cw1_ref[...]) + cb1_ref[...]


def _fold(w2, cw1, b2, cb1):
    return pl.pallas_call(
        _fold_body,
        out_shape=[
            jax.ShapeDtypeStruct((D, D), jnp.float32),
            jax.ShapeDtypeStruct((1, D), jnp.float32),
        ],
    )(w2, cw1, b2, cb1)


def _sc_gather_body(p_hbm, q_hbm, src_hbm, dst_hbm, gs_hbm, gd_hbm,
                    si_v, di_v, bs0_v, bd0_v, bs1_v, bd1_v,
                    sp0, sq0, sp1, sq1):
    wid = lax.axis_index("s") * NC + lax.axis_index("c")
    base = wid * EPW
    pltpu.sync_copy(src_hbm.at[pl.ds(base, EPW)], si_v)
    pltpu.sync_copy(dst_hbm.at[pl.ds(base, EPW)], di_v)

    def fire(c, bs, bd, sp, sq):
        off = c * CHG
        pltpu.async_copy(p_hbm.at[si_v.at[pl.ds(off, CHG)]], bs, sp)
        pltpu.async_copy(q_hbm.at[di_v.at[pl.ds(off, CHG)]], bd, sq)

    def drain_write(c, bs, bd, sp, sq):
        pltpu.make_async_copy(p_hbm.at[pl.ds(0, CHG)], bs, sp).wait()
        pltpu.make_async_copy(q_hbm.at[pl.ds(0, CHG)], bd, sq).wait()
        off = c * CHG
        pltpu.sync_copy(bs, gs_hbm.at[pl.ds(base + off, CHG)])
        pltpu.sync_copy(bd, gd_hbm.at[pl.ds(base + off, CHG)])

    fire(0, bs0_v, bd0_v, sp0, sq0)

    def stream_body(k, carry):
        c0 = 2 * k
        fire(c0 + 1, bs1_v, bd1_v, sp1, sq1)
        drain_write(c0, bs0_v, bd0_v, sp0, sq0)
        fire(jnp.minimum(c0 + 2, NCHG - 1), bs0_v, bd0_v, sp0, sq0)
        drain_write(c0 + 1, bs1_v, bd1_v, sp1, sq1)
        return carry

    lax.fori_loop(0, NCHG // 2, stream_body, 0)
    pltpu.make_async_copy(p_hbm.at[pl.ds(0, CHG)], bs0_v, sp0).wait()
    pltpu.make_async_copy(q_hbm.at[pl.ds(0, CHG)], bd0_v, sq0).wait()


def _sc_gather(ppack, qpack, src, dst):
    mesh = plsc.VectorSubcoreMesh(core_axis_name="c", subcore_axis_name="s")
    f = functools.partial(
        pl.kernel,
        mesh=mesh,
        compiler_params=pltpu.CompilerParams(needs_layout_passes=False),
        out_type=[
            jax.ShapeDtypeStruct((EPAD, D), jnp.int32),
            jax.ShapeDtypeStruct((EPAD, D), jnp.int32),
        ],
        scratch_types=[
            pltpu.VMEM((EPW,), jnp.int32),
            pltpu.VMEM((EPW,), jnp.int32),
            pltpu.VMEM((CHG, D), jnp.int32),
            pltpu.VMEM((CHG, D), jnp.int32),
            pltpu.VMEM((CHG, D), jnp.int32),
            pltpu.VMEM((CHG, D), jnp.int32),
            pltpu.SemaphoreType.DMA,
            pltpu.SemaphoreType.DMA,
            pltpu.SemaphoreType.DMA,
            pltpu.SemaphoreType.DMA,
        ],
    )(_sc_gather_body)
    return f(ppack, qpack, src, dst)


def _edge_body(gs_ref, gd_ref, w1c_ref, m_ref, bu_ref, cw2_ref,
               cb2_ref, t_ref, b_ref):
    i = pl.program_id(0)
    fa_s, fb_s = _unpack2(gs_ref[...])
    fa_d, fb_d = _unpack2(gd_ref[...])
    feat = fa_s + fa_d
    xrel = (fb_s + fb_d)[:, 0:3]
    r = feat.shape[0]
    mag = jnp.sum(xrel * xrel, axis=1, keepdims=True)
    rbf = jnp.concatenate([jnp.exp(-mag / s) for s in _SIG], axis=1)
    pre = feat + _mm(rbf, w1c_ref[...])
    t = _leaky(pre)
    u = _leaky(_mm(t, m_ref[...]) + bu_ref[...])
    coef = _mm(u, cw2_ref[...]) + cb2_ref[...]
    valid = (lax.broadcasted_iota(jnp.int32, (r, 1), 0) + i * r) < E
    t_ref[...] = jnp.where(valid, t, 0.0)
    xm = jnp.where(valid, xrel * coef, 0.0)
    cnt = jnp.where(valid, jnp.ones((r, 1), jnp.float32), 0.0)
    zpad = jnp.zeros((r, D - 4), jnp.float32)
    b_ref[...] = jnp.concatenate([xm, cnt, zpad], axis=1)


def _edge(gs, gd, w1c, m, bu, cw2, cb2):
    R = 8192
    g = EPAD // R
    return pl.pallas_call(
        _edge_body,
        grid=(g,),
        in_specs=[
            pl.BlockSpec((R, D), lambda i: (i, 0)),
            pl.BlockSpec((R, D), lambda i: (i, 0)),
            pl.BlockSpec((15, D), lambda i: (0, 0)),
            pl.BlockSpec((D, D), lambda i: (0, 0)),
            pl.BlockSpec((1, D), lambda i: (0, 0)),
            pl.BlockSpec((D, 1), lambda i: (0, 0)),
            pl.BlockSpec((1, 1), lambda i: (0, 0)),
        ],
        out_specs=[
            pl.BlockSpec((R, D), lambda i: (i, 0)),
            pl.BlockSpec((R, D), lambda i: (i, 0)),
        ],
        out_shape=[
            jax.ShapeDtypeStruct((EPAD, D), jnp.float32),
            jax.ShapeDtypeStruct((EPAD, D), jnp.float32),
        ],
    )(gs, gd, w1c, m, bu, cw2, cb2)


def _sc_scatter_body(pay_hbm, dst2_hbm, zero_hbm, out_hbm,
                     di_v, buf0_v, buf1_v, acc_sh, sl0, sl1):
    cid = lax.axis_index("c")
    sid = lax.axis_index("s")
    wid = sid * NC + cid
    base = wid * EPW

    pltpu.sync_copy(zero_hbm.at[pl.ds(sid * 640, 640)],
                    acc_sh.at[pl.ds(sid * 640, 640)])
    pltpu.sync_copy(dst2_hbm.at[pl.ds(wid * NCHUNK, NCHUNK)], di_v)
    plsc.subcore_barrier()

    def fire(c, buf, sem):
        pltpu.async_copy(pay_hbm.at[pl.ds(base + c * CH, CH)], buf, sem)

    def drain_add(c, buf, sem):
        pltpu.make_async_copy(pay_hbm.at[pl.ds(0, CH)], buf, sem).wait()
        pltpu.sync_copy(buf, acc_sh.at[di_v.at[c]], add=True)

    fire(0, buf0_v, sl0)

    def body(k, carry):
        c0 = 2 * k
        fire(c0 + 1, buf1_v, sl1)
        drain_add(c0, buf0_v, sl0)
        fire(jnp.minimum(c0 + 2, NCHUNK - 1), buf0_v, sl0)
        drain_add(c0 + 1, buf1_v, sl1)
        return carry

    lax.fori_loop(0, NCHUNK // 2, body, 0)
    pltpu.make_async_copy(pay_hbm.at[pl.ds(0, CH)], buf0_v, sl0).wait()
    plsc.subcore_barrier()
    pltpu.sync_copy(acc_sh.at[pl.ds(sid * 640, 640)],
                    out_hbm.at[cid, pl.ds(sid * 640, 640)])


def _sc_scatter(pay, dst2, zero):
    mesh = plsc.VectorSubcoreMesh(core_axis_name="c", subcore_axis_name="s")
    f = functools.partial(
        pl.kernel,
        mesh=mesh,
        compiler_params=pltpu.CompilerParams(needs_layout_passes=False),
        out_type=jax.ShapeDtypeStruct((NC, NPAD, D), jnp.float32),
        scratch_types=[
            pltpu.VMEM((NCHUNK, CH), jnp.int32),
            pltpu.VMEM((CH, D), jnp.float32),
            pltpu.VMEM((CH, D), jnp.float32),
            pltpu.VMEM_SHARED((NPAD, D), jnp.float32),
            pltpu.SemaphoreType.DMA,
            pltpu.SemaphoreType.DMA,
        ],
    )(_sc_scatter_body)
    return f(pay, dst2, zero)


def _combine_body(at_ref, ab_ref, w2_ref, b2_ref, x_ref, m_ref):
    a_t = at_ref[0] + at_ref[1]
    a_b = ab_ref[0] + ab_ref[1]
    cnt = a_b[:, 3:4]
    denom = jnp.maximum(cnt, 1.0)
    has = jnp.where(cnt > 0, 1.0, 0.0)
    x_ref[...] = a_b[:, 0:3] / denom
    m_ref[...] = _mm(a_t / denom, w2_ref[...]) + b2_ref[...] * has


def _combine(acc_t, acc_b, w2, b2):
    R = 400
    g = N // R
    return pl.pallas_call(
        _combine_body,
        grid=(g,),
        in_specs=[
            pl.BlockSpec((NC, R, D), lambda i: (0, i, 0)),
            pl.BlockSpec((NC, R, D), lambda i: (0, i, 0)),
            pl.BlockSpec((D, D), lambda i: (0, 0)),
            pl.BlockSpec((1, D), lambda i: (0, 0)),
        ],
        out_specs=[
            pl.BlockSpec((R, 3), lambda i: (i, 0)),
            pl.BlockSpec((R, D), lambda i: (i, 0)),
        ],
        out_shape=[
            jax.ShapeDtypeStruct((N, 3), jnp.float32),
            jax.ShapeDtypeStruct((N, D), jnp.float32),
        ],
    )(acc_t, acc_b, w2, b2)


def _flash_body(q_ref, k_ref, v_ref, o_ref, m_scr, l_scr, acc_scr):
    j = pl.program_id(1)
    nj = pl.num_programs(1)

    @pl.when(j == 0)
    def _init():
        m_scr[...] = jnp.full_like(m_scr, -1e30)
        l_scr[...] = jnp.zeros_like(l_scr)
        acc_scr[...] = jnp.zeros_like(acc_scr)

    q = q_ref[...]
    k = k_ref[...]
    s = lax.dot_general(q, k, (((1,), (1,)), ((), ())),
                        preferred_element_type=jnp.float32)
    s = s.astype(jnp.float32)
    m_old = m_scr[:, :1]
    m_new = jnp.maximum(m_old, jnp.max(s, axis=1, keepdims=True))
    p = jnp.exp(s - m_new)
    corr = jnp.exp(m_old - m_new)
    l_new = l_scr[:, :1] * corr + jnp.sum(p, axis=1, keepdims=True)
    acc_scr[...] = acc_scr[...] * corr + _mm(p.astype(jnp.bfloat16),
                                             v_ref[...]).astype(jnp.float32)
    m_scr[...] = jnp.broadcast_to(m_new, m_scr.shape)
    l_scr[...] = jnp.broadcast_to(l_new, l_scr.shape)

    @pl.when(j == nj - 1)
    def _out():
        o_ref[...] = acc_scr[...] / l_scr[:, :1]


def _flash(q, k, v):
    BQ, BK = 2000, 2000
    gi, gj = N // BQ, N // BK
    return pl.pallas_call(
        _flash_body,
        grid=(gi, gj),
        in_specs=[
            pl.BlockSpec((BQ, D), lambda i, j: (i, 0)),
            pl.BlockSpec((BK, D), lambda i, j: (j, 0)),
            pl.BlockSpec((BK, D), lambda i, j: (j, 0)),
        ],
        out_specs=pl.BlockSpec((BQ, D), lambda i, j: (i, 0)),
        out_shape=jax.ShapeDtypeStruct((N, D), jnp.float32),
        scratch_shapes=[
            pltpu.VMEM((BQ, D), jnp.float32),
            pltpu.VMEM((BQ, D), jnp.float32),
            pltpu.VMEM((BQ, D), jnp.float32),
        ],
        compiler_params=pltpu.CompilerParams(
            dimension_semantics=("arbitrary", "arbitrary")),
    )(q, k, v)


def _asm_body(xl_ref, ml_ref, cl_ref, xr_ref, mr_ref, cr_ref, o_ref):
    o_ref[...] = jnp.concatenate(
        [xl_ref[...], ml_ref[...], cl_ref[...],
         xr_ref[...], mr_ref[...], cr_ref[...]], axis=1)


def _assemble(xl, ml, cl, xr, mr, cr):
    R = 400
    g = N // R
    x3 = pl.BlockSpec((R, 3), lambda i: (i, 0))
    fd = pl.BlockSpec((R, D), lambda i: (i, 0))
    return pl.pallas_call(
        _asm_body,
        grid=(g,),
        in_specs=[x3, fd, fd, x3, fd, fd],
        out_specs=pl.BlockSpec((R, 4 * D + 6), lambda i: (i, 0)),
        out_shape=jax.ShapeDtypeStruct((N, 4 * D + 6), jnp.float32),
    )(xl, ml, cl, xr, mr, cr)


def _per_graph(coors, h, ei, wa, wb, b1, w1c, m, bu, cw2, cb2, w2, b2,
               attq, attk, attv, zero, pad_i):
    ppack, qpack, oq, ok, ov = _proj(h, coors, wa, wb, b1, attq, attk, attv)
    src = jnp.concatenate([ei[0].astype(jnp.int32), pad_i])
    dst = jnp.concatenate([ei[1].astype(jnp.int32), pad_i])
    gs, gd = _sc_gather(ppack, qpack, src, dst)
    pay_t, pay_b = _edge(gs, gd, w1c, m, bu, cw2, cb2)
    dst2 = jnp.reshape(dst, (EPAD // CH, CH))
    acc_t = _sc_scatter(pay_t, dst2, zero)
    acc_b = _sc_scatter(pay_b, dst2, zero)
    x_upd, aggr = _combine(acc_t, acc_b, w2, b2)
    return x_upd, aggr, oq, ok, ov


def kernel(coors_lig, h_feats_ligand, coors_rec, h_feats_receptor,
           edge_W1, edge_b1, edge_W2, edge_b2,
           att_Q, att_K, att_V,
           coors_W1, coors_b1, coors_W2, coors_b2,
           lig_edge_index, rec_edge_index):
    wa = edge_W1[:D]
    wb = edge_W1[D:2 * D]
    w1c = edge_W1[2 * D:]
    b1 = edge_b1.reshape(1, D)
    b2 = edge_b2.reshape(1, D)
    cb1 = coors_b1.reshape(1, D)
    cb2 = coors_b2.reshape(1, 1)
    m, bu = _fold(edge_W2, coors_W1, b2, cb1)
    zero = jnp.zeros((NPAD, D), jnp.float32)
    pad_i = jnp.zeros((EPAD - E,), jnp.int32)

    xl, ml, ql, kl, vl = _per_graph(
        coors_lig, h_feats_ligand, lig_edge_index,
        wa, wb, b1, w1c, m, bu, coors_W2, cb2, edge_W2, b2,
        att_Q, att_K, att_V, zero, pad_i)
    xr, mr, qr, kr, vr = _per_graph(
        coors_rec, h_feats_receptor, rec_edge_index,
        wa, wb, b1, w1c, m, bu, coors_W2, cb2, edge_W2, b2,
        att_Q, att_K, att_V, zero, pad_i)

    cl = _flash(ql, kr, vr)
    cr = _flash(qr, kl, vl)
    return _assemble(xl, ml, cl, xr, mr, cr)
